# baseline (device time: 51823 ns/iter reference)
import jax
import jax.numpy as jnp
from jax import lax
from jax.experimental import pallas as pl
from jax.experimental.pallas import tpu as pltpu

N_DEV = 4
B, Sq, Skv, Hq, Dh = 2, 256, 1024, 4, 64
HD = Hq * Dh
D = 512
S_SH = Skv // N_DEV


def kernel(x, Wq, K_ext, V_ext, Wo):
    K2 = K_ext.reshape(B, S_SH, HD)
    V2 = V_ext.reshape(B, S_SH, HD)

    def body(x_ref, wq_ref, k_ref, v_ref, wo_ref, out_ref,
             k_full, v_full, comm_k, comm_v,
             ksend, krecv, vsend, vrecv):
        my = lax.axis_index("i")
        left = lax.rem(my + N_DEV - 1, N_DEV)
        right = lax.rem(my + 1, N_DEV)

        barrier = pltpu.get_barrier_semaphore()
        for nbr in (left, right):
            pl.semaphore_signal(barrier, inc=1, device_id=(nbr,),
                                device_id_type=pl.DeviceIdType.MESH)
        pl.semaphore_wait(barrier, 2)

        k_full[:, pl.ds(my * S_SH, S_SH), :] = k_ref[...]
        v_full[:, pl.ds(my * S_SH, S_SH), :] = v_ref[...]
        comm_k[0] = k_ref[...]
        comm_v[0] = v_ref[...]

        for h in range(N_DEV - 1):
            s, r = h % 2, (h + 1) % 2
            rk = pltpu.make_async_remote_copy(
                src_ref=comm_k.at[s], dst_ref=comm_k.at[r],
                send_sem=ksend.at[s], recv_sem=krecv.at[r],
                device_id=(right,), device_id_type=pl.DeviceIdType.MESH)
            rv = pltpu.make_async_remote_copy(
                src_ref=comm_v.at[s], dst_ref=comm_v.at[r],
                send_sem=vsend.at[s], recv_sem=vrecv.at[r],
                device_id=(right,), device_id_type=pl.DeviceIdType.MESH)
            rk.start()
            rv.start()
            rk.wait()
            rv.wait()
            origin = lax.rem(my - (h + 1) + 2 * N_DEV, N_DEV)
            k_full[:, pl.ds(origin * S_SH, S_SH), :] = comm_k[r]
            v_full[:, pl.ds(origin * S_SH, S_SH), :] = comm_v[r]

        qi = lax.broadcasted_iota(jnp.int32, (Sq, Skv), 0)
        ki = lax.broadcasted_iota(jnp.int32, (Sq, Skv), 1)
        mask = (jnp.abs(qi - ki) <= 128) | (ki < 32) | (qi < 32)
        neg = jnp.float32(-1e9)

        for b in range(B):
            q_b = jnp.dot(x_ref[b], wq_ref[...],
                          preferred_element_type=jnp.float32)
            ctx_parts = []
            for hh in range(Hq):
                qh = q_b[:, hh * Dh:(hh + 1) * Dh]
                kh = k_full[b, :, hh * Dh:(hh + 1) * Dh]
                scores = lax.dot_general(
                    qh, kh, (((1,), (1,)), ((), ())),
                    preferred_element_type=jnp.float32) * 0.125
                scores = jnp.where(mask, scores, neg)
                m = jnp.max(scores, axis=-1, keepdims=True)
                w = jnp.exp(scores - m)
                w = w / jnp.sum(w, axis=-1, keepdims=True)
                vh = v_full[b, :, hh * Dh:(hh + 1) * Dh]
                ctx_parts.append(
                    jnp.dot(w, vh, preferred_element_type=jnp.float32))
            ctx = jnp.concatenate(ctx_parts, axis=1)
            out_ref[b] = jnp.dot(ctx, wo_ref[...],
                                 preferred_element_type=jnp.float32)

    return pl.pallas_call(
        body,
        out_shape=jax.ShapeDtypeStruct((B, Sq, D), jnp.float32),
        in_specs=[pl.BlockSpec(memory_space=pltpu.VMEM)] * 5,
        out_specs=pl.BlockSpec(memory_space=pltpu.VMEM),
        scratch_shapes=[
            pltpu.VMEM((B, Skv, HD), jnp.float32),
            pltpu.VMEM((B, Skv, HD), jnp.float32),
            pltpu.VMEM((2, B, S_SH, HD), jnp.float32),
            pltpu.VMEM((2, B, S_SH, HD), jnp.float32),
            pltpu.SemaphoreType.DMA((2,)),
            pltpu.SemaphoreType.DMA((2,)),
            pltpu.SemaphoreType.DMA((2,)),
            pltpu.SemaphoreType.DMA((2,)),
        ],
        compiler_params=pltpu.CompilerParams(collective_id=0),
    )(x, Wq, K2, V2, Wo)


# device time: 24457 ns/iter; 2.1189x vs baseline; 2.1189x over previous
import jax
import jax.numpy as jnp
from jax import lax
from jax.experimental import pallas as pl
from jax.experimental.pallas import tpu as pltpu

N_DEV = 4
B, Sq, Skv, Hq, Dh = 2, 256, 1024, 4, 64
HD = Hq * Dh
D = 512
S_SH = Skv // N_DEV
F32 = jnp.float32


def kernel(x, Wq, K_ext, V_ext, Wo):
    K2 = K_ext.reshape(B, S_SH, HD)
    V2 = V_ext.reshape(B, S_SH, HD)

    def body(x_ref, wq_ref, k_ref, v_ref, wo_ref, out_ref,
             pbuf, abuf, rbuf, lp, al, rl,
             csend, crecv, lsend, lrecv):
        my = lax.axis_index("i")
        left = lax.rem(my + N_DEV - 1, N_DEV)
        right = lax.rem(my + 1, N_DEV)
        p1 = my ^ 1
        p2 = 3 - my

        koff = my * S_SH
        qi = lax.broadcasted_iota(jnp.int32, (Sq, S_SH), 0)
        kig = lax.broadcasted_iota(jnp.int32, (Sq, S_SH), 1) + koff
        mask = (jnp.abs(qi - kig) <= 128) | (kig < 32) | (qi < 32)

        for b in range(B):
            q_b = jnp.dot(x_ref[b], wq_ref[...],
                          preferred_element_type=F32)
            lcols = []
            for h in range(Hq):
                qh = q_b[:, h * Dh:(h + 1) * Dh]
                kh = k_ref[b, :, h * Dh:(h + 1) * Dh]
                s = lax.dot_general(
                    qh, kh, (((1,), (1,)), ((), ())),
                    preferred_element_type=F32) * 0.125
                w = jnp.where(mask, jnp.exp(s), 0.0)
                vh = v_ref[b, :, h * Dh:(h + 1) * Dh]
                pbuf[b, :, h * Dh:(h + 1) * Dh] = jnp.dot(
                    w, vh, preferred_element_type=F32)
                lcols.append(jnp.sum(w, axis=1, keepdims=True))
            l_b = jnp.concatenate(
                lcols + [jnp.zeros((Sq, 8 - Hq), F32)], axis=1)
            lp[b] = jnp.transpose(l_b)

        barrier = pltpu.get_barrier_semaphore()
        for nbr in (left, right):
            pl.semaphore_signal(barrier, inc=1, device_id=(nbr,),
                                device_id_type=pl.DeviceIdType.MESH)
        pl.semaphore_wait(barrier, 2)

        rc1 = pltpu.make_async_remote_copy(
            src_ref=pbuf, dst_ref=rbuf.at[0],
            send_sem=csend.at[0], recv_sem=crecv.at[0],
            device_id=(p1,), device_id_type=pl.DeviceIdType.MESH)
        rl1 = pltpu.make_async_remote_copy(
            src_ref=lp, dst_ref=rl.at[0],
            send_sem=lsend.at[0], recv_sem=lrecv.at[0],
            device_id=(p1,), device_id_type=pl.DeviceIdType.MESH)
        rc1.start()
        rl1.start()
        rc1.wait()
        rl1.wait()
        abuf[...] = pbuf[...] + rbuf[0]
        al[...] = lp[...] + rl[0]

        rc2 = pltpu.make_async_remote_copy(
            src_ref=abuf, dst_ref=rbuf.at[1],
            send_sem=csend.at[1], recv_sem=crecv.at[1],
            device_id=(p2,), device_id_type=pl.DeviceIdType.MESH)
        rl2 = pltpu.make_async_remote_copy(
            src_ref=al, dst_ref=rl.at[1],
            send_sem=lsend.at[1], recv_sem=lrecv.at[1],
            device_id=(p2,), device_id_type=pl.DeviceIdType.MESH)
        rc2.start()
        rl2.start()
        rc2.wait()
        rl2.wait()

        for b in range(B):
            ctx = abuf[b] + rbuf[1, b]
            l_b = jnp.transpose(al[b] + rl[1, b])
            parts = []
            for h in range(Hq):
                parts.append(ctx[:, h * Dh:(h + 1) * Dh]
                             / l_b[:, h:h + 1])
            ctx_n = jnp.concatenate(parts, axis=1)
            out_ref[b] = jnp.dot(ctx_n, wo_ref[...],
                                 preferred_element_type=F32)

    return pl.pallas_call(
        body,
        out_shape=jax.ShapeDtypeStruct((B, Sq, D), jnp.float32),
        in_specs=[pl.BlockSpec(memory_space=pltpu.VMEM)] * 5,
        out_specs=pl.BlockSpec(memory_space=pltpu.VMEM),
        scratch_shapes=[
            pltpu.VMEM((B, Sq, HD), F32),
            pltpu.VMEM((B, Sq, HD), F32),
            pltpu.VMEM((2, B, Sq, HD), F32),
            pltpu.VMEM((B, 8, Sq), F32),
            pltpu.VMEM((B, 8, Sq), F32),
            pltpu.VMEM((2, B, 8, Sq), F32),
            pltpu.SemaphoreType.DMA((2,)),
            pltpu.SemaphoreType.DMA((2,)),
            pltpu.SemaphoreType.DMA((2,)),
            pltpu.SemaphoreType.DMA((2,)),
        ],
        compiler_params=pltpu.CompilerParams(collective_id=0),
    )(x, Wq, K2, V2, Wo)


# device time: 18802 ns/iter; 2.7562x vs baseline; 1.3008x over previous
import jax
import jax.numpy as jnp
from jax import lax
from jax.experimental import pallas as pl
from jax.experimental.pallas import tpu as pltpu

N_DEV = 4
B, Sq, Skv, Hq, Dh = 2, 256, 1024, 4, 64
HD = Hq * Dh
D = 512
S_SH = Skv // N_DEV
F32 = jnp.float32


def kernel(x, Wq, K_ext, V_ext, Wo):
    K2 = K_ext.reshape(B, S_SH, HD)
    V2 = V_ext.reshape(B, S_SH, HD)

    def body(x_ref, wq_ref, k_ref, v_ref, wo_ref, out_ref,
             pbuf, abuf, abuf_bf, rbuf, lp, al, rl,
             csend, crecv, lsend, lrecv):
        my = lax.axis_index("i")
        left = lax.rem(my + N_DEV - 1, N_DEV)
        right = lax.rem(my + 1, N_DEV)
        p1 = my ^ 1
        p2 = 3 - my

        koff = my * S_SH
        qi = lax.broadcasted_iota(jnp.int32, (Sq, S_SH), 0)
        kig = lax.broadcasted_iota(jnp.int32, (Sq, S_SH), 1) + koff
        mask = (jnp.abs(qi - kig) <= 128) | (kig < 32) | (qi < 32)

        for b in range(B):
            q_b = jnp.dot(x_ref[b], wq_ref[...],
                          preferred_element_type=F32)
            lcols = []
            for h in range(Hq):
                qh = q_b[:, h * Dh:(h + 1) * Dh]
                kh = k_ref[b, :, h * Dh:(h + 1) * Dh]
                s = lax.dot_general(
                    qh, kh, (((1,), (1,)), ((), ())),
                    preferred_element_type=F32) * 0.125
                w = jnp.where(mask, jnp.exp(s), 0.0)
                vh = v_ref[b, :, h * Dh:(h + 1) * Dh]
                pbuf[b, :, h * Dh:(h + 1) * Dh] = jnp.dot(
                    w, vh, preferred_element_type=F32).astype(jnp.bfloat16)
                lcols.append(jnp.sum(w, axis=1, keepdims=True))
            l_b = jnp.concatenate(
                lcols + [jnp.zeros((Sq, 8 - Hq), F32)], axis=1)
            lp[b] = jnp.transpose(l_b)

        barrier = pltpu.get_barrier_semaphore()
        for nbr in (left, right):
            pl.semaphore_signal(barrier, inc=1, device_id=(nbr,),
                                device_id_type=pl.DeviceIdType.MESH)
        pl.semaphore_wait(barrier, 2)

        rc1 = pltpu.make_async_remote_copy(
            src_ref=pbuf, dst_ref=rbuf.at[0],
            send_sem=csend.at[0], recv_sem=crecv.at[0],
            device_id=(p1,), device_id_type=pl.DeviceIdType.MESH)
        rl1 = pltpu.make_async_remote_copy(
            src_ref=lp, dst_ref=rl.at[0],
            send_sem=lsend.at[0], recv_sem=lrecv.at[0],
            device_id=(p1,), device_id_type=pl.DeviceIdType.MESH)
        rc1.start()
        rl1.start()
        rc1.wait()
        rl1.wait()
        abuf[...] = pbuf[...].astype(F32) + rbuf[0].astype(F32)
        abuf_bf[...] = abuf[...].astype(jnp.bfloat16)
        al[...] = lp[...] + rl[0]

        rc2 = pltpu.make_async_remote_copy(
            src_ref=abuf_bf, dst_ref=rbuf.at[1],
            send_sem=csend.at[1], recv_sem=crecv.at[1],
            device_id=(p2,), device_id_type=pl.DeviceIdType.MESH)
        rl2 = pltpu.make_async_remote_copy(
            src_ref=al, dst_ref=rl.at[1],
            send_sem=lsend.at[1], recv_sem=lrecv.at[1],
            device_id=(p2,), device_id_type=pl.DeviceIdType.MESH)
        rc2.start()
        rl2.start()
        rc2.wait()
        rl2.wait()

        for b in range(B):
            ctx = abuf[b] + rbuf[1, b].astype(F32)
            l_b = jnp.transpose(al[b] + rl[1, b])
            parts = []
            for h in range(Hq):
                parts.append(ctx[:, h * Dh:(h + 1) * Dh]
                             / l_b[:, h:h + 1])
            ctx_n = jnp.concatenate(parts, axis=1)
            out_ref[b] = jnp.dot(ctx_n, wo_ref[...],
                                 preferred_element_type=F32)

    return pl.pallas_call(
        body,
        out_shape=jax.ShapeDtypeStruct((B, Sq, D), jnp.float32),
        in_specs=[pl.BlockSpec(memory_space=pltpu.VMEM)] * 5,
        out_specs=pl.BlockSpec(memory_space=pltpu.VMEM),
        scratch_shapes=[
            pltpu.VMEM((B, Sq, HD), jnp.bfloat16),
            pltpu.VMEM((B, Sq, HD), F32),
            pltpu.VMEM((B, Sq, HD), jnp.bfloat16),
            pltpu.VMEM((2, B, Sq, HD), jnp.bfloat16),
            pltpu.VMEM((B, 8, Sq), F32),
            pltpu.VMEM((B, 8, Sq), F32),
            pltpu.VMEM((2, B, 8, Sq), F32),
            pltpu.SemaphoreType.DMA((2,)),
            pltpu.SemaphoreType.DMA((2,)),
            pltpu.SemaphoreType.DMA((2,)),
            pltpu.SemaphoreType.DMA((2,)),
        ],
        compiler_params=pltpu.CompilerParams(collective_id=0),
    )(x, Wq, K2, V2, Wo)


# device time: 15852 ns/iter; 3.2692x vs baseline; 1.1861x over previous
import jax
import jax.numpy as jnp
from jax import lax
from jax.experimental import pallas as pl
from jax.experimental.pallas import tpu as pltpu

N_DEV = 4
B, Sq, Skv, Hq, Dh = 2, 256, 1024, 4, 64
HD = Hq * Dh
D = 512
S_SH = Skv // N_DEV
R = Sq + 8
F32 = jnp.float32
BF16 = jnp.bfloat16


def kernel(x, Wq, K_ext, V_ext, Wo):
    K2 = K_ext.reshape(B, S_SH, HD)
    V2 = V_ext.reshape(B, S_SH, HD)

    def body(x_ref, wq_ref, k_ref, v_ref, wo_ref, out_ref,
             pbuf, abuf, abuf_bf, rbuf, csend, crecv):
        my = lax.axis_index("i")
        left = lax.rem(my + N_DEV - 1, N_DEV)
        right = lax.rem(my + 1, N_DEV)
        p1 = my ^ 1
        p2 = 3 - my

        barrier = pltpu.get_barrier_semaphore()
        for nbr in (left, right):
            pl.semaphore_signal(barrier, inc=1, device_id=(nbr,),
                                device_id_type=pl.DeviceIdType.MESH)
        pl.semaphore_wait(barrier, 2)

        koff = my * S_SH
        qi = lax.broadcasted_iota(jnp.int32, (Sq, S_SH), 0)
        kig = lax.broadcasted_iota(jnp.int32, (Sq, S_SH), 1) + koff
        mask = (jnp.abs(qi - kig) <= 128) | (kig < 32) | (qi < 32)

        def rdma(stage, b, src, partner):
            return pltpu.make_async_remote_copy(
                src_ref=src.at[b], dst_ref=rbuf.at[stage, b],
                send_sem=csend.at[stage, b], recv_sem=crecv.at[stage, b],
                device_id=(partner,), device_id_type=pl.DeviceIdType.MESH)

        rc1 = [None, None]
        for b in range(B):
            q_b = jnp.dot(x_ref[b], wq_ref[...],
                          preferred_element_type=F32)
            lcols = []
            for h in range(Hq):
                qh = q_b[:, h * Dh:(h + 1) * Dh]
                kh = k_ref[b, :, h * Dh:(h + 1) * Dh]
                s = lax.dot_general(
                    qh, kh, (((1,), (1,)), ((), ())),
                    preferred_element_type=F32) * 0.125
                w = jnp.where(mask, jnp.exp(s), 0.0)
                vh = v_ref[b, :, h * Dh:(h + 1) * Dh]
                pbuf[b, :Sq, h * Dh:(h + 1) * Dh] = jnp.dot(
                    w, vh, preferred_element_type=F32).astype(BF16)
                lcols.append(jnp.sum(w, axis=1, keepdims=True))
            l_b = jnp.concatenate(
                lcols + [jnp.zeros((Sq, 8 - Hq), F32)], axis=1)
            pbuf[b, Sq:, :] = jnp.transpose(l_b).astype(BF16)
            rc1[b] = rdma(0, b, pbuf, p1)
            rc1[b].start()

        rc2 = [None, None]
        for b in range(B):
            rc1[b].wait()
            abuf[b] = pbuf[b].astype(F32) + rbuf[0, b].astype(F32)
            abuf_bf[b] = abuf[b].astype(BF16)
            rc2[b] = rdma(1, b, abuf_bf, p2)
            rc2[b].start()

        for b in range(B):
            rc2[b].wait()
            ctx = abuf[b, :Sq, :] + rbuf[1, b, :Sq, :].astype(F32)
            l_t = abuf[b, Sq:, :] + rbuf[1, b, Sq:, :].astype(F32)
            l_b = jnp.transpose(l_t)
            parts = []
            for h in range(Hq):
                parts.append(ctx[:, h * Dh:(h + 1) * Dh]
                             / l_b[:, h:h + 1])
            ctx_n = jnp.concatenate(parts, axis=1)
            out_ref[b] = jnp.dot(ctx_n, wo_ref[...],
                                 preferred_element_type=F32)

    return pl.pallas_call(
        body,
        out_shape=jax.ShapeDtypeStruct((B, Sq, D), jnp.float32),
        in_specs=[pl.BlockSpec(memory_space=pltpu.VMEM)] * 5,
        out_specs=pl.BlockSpec(memory_space=pltpu.VMEM),
        scratch_shapes=[
            pltpu.VMEM((B, R, HD), BF16),
            pltpu.VMEM((B, R, HD), F32),
            pltpu.VMEM((B, R, HD), BF16),
            pltpu.VMEM((2, B, R, HD), BF16),
            pltpu.SemaphoreType.DMA((2, B)),
            pltpu.SemaphoreType.DMA((2, B)),
        ],
        compiler_params=pltpu.CompilerParams(collective_id=0),
    )(x, Wq, K2, V2, Wo)
